# baseline (speedup 1.0000x reference)
import functools

import jax
import jax.numpy as jnp
from jax import lax
from jax.experimental import pallas as pl
from jax.experimental.pallas import tpu as pltpu
from jax.experimental.pallas import tpu_sc as plsc

_EPS = 1e-12


_NUM_CORES = 2
_NUM_SUBCORES = 16
_NUM_WORKERS = _NUM_CORES * _NUM_SUBCORES


def _sc_gather_bf16(table_u32, ids_flat, chunk, piece0, npiece):
    h = table_u32.shape[1]
    hw = h // 2
    ngrp = h // 32
    tok_per_w = npiece // _NUM_WORKERS
    n_chunks = tok_per_w // chunk
    mesh = plsc.VectorSubcoreMesh(core_axis_name="c", subcore_axis_name="s")

    @functools.partial(
        pl.kernel,
        mesh=mesh,
        out_type=jax.ShapeDtypeStruct((npiece, hw), jnp.uint32),
        scratch_types=[
            pltpu.VMEM((tok_per_w,), jnp.int32),
            pltpu.VMEM((chunk, h), jnp.uint32),
            pltpu.VMEM((chunk, h), jnp.uint32),
            pltpu.SemaphoreType.DMA,
            pltpu.SemaphoreType.DMA,
        ],
    )
    def gather_kernel(table_hbm, idx_hbm, out_hbm, idx_v, buf0, buf1, sem0,
                      sem1):
        wid = lax.axis_index("s") * _NUM_CORES + lax.axis_index("c")
        base = wid * tok_per_w
        pltpu.sync_copy(idx_hbm.at[pl.ds(piece0 + base, tok_per_w)], idx_v)

        bufs = (buf0, buf1)
        sems = (sem0, sem1)

        pltpu.async_copy(table_hbm.at[idx_v.at[pl.ds(0, chunk)]], buf0, sem0)

        def body(i, _):
            for p in range(2):
                nxt = i + 1

                @pl.when(jnp.logical_and(nxt % 2 == p, nxt < n_chunks))
                def _():
                    pltpu.async_copy(
                        table_hbm.at[idx_v.at[pl.ds(nxt * chunk, chunk)]],
                        bufs[p],
                        sems[p],
                    )

            for p in range(2):

                @pl.when(i % 2 == p)
                def _():
                    pltpu.make_async_copy(
                        table_hbm.at[idx_v.at[pl.ds(i * chunk, chunk)]],
                        bufs[p],
                        sems[p],
                    ).wait()

                    def row_body(r, _):
                        buf_r = bufs[p].at[r]
                        for g in range(ngrp):
                            a = buf_r[pl.ds(32 * g, 16)]
                            b = buf_r[pl.ds(32 * g + 16, 16)]
                            lo = (a + 0x8000) >> 16
                            hi = (b + 0x8000) & jnp.uint32(0xFFFF0000)
                            buf_r[pl.ds(16 * g, 16)] = lo | hi
                        return 0

                    lax.fori_loop(0, chunk, row_body, 0, unroll=2)
                    pltpu.sync_copy(
                        bufs[p].at[:, pl.ds(0, hw)],
                        out_hbm.at[pl.ds(base + i * chunk, chunk)],
                    )

            return 0

        lax.fori_loop(0, n_chunks, body, 0)

    return gather_kernel(table_u32, ids_flat)


def _ln_body(carry_ref, w_ref, perm_ref, tt_ref, pos_ref, type_ref, lnw_ref,
             lnb_ref, o_ref):
    del carry_ref
    xs = w_ref[0]
    x = lax.dot_general(
        xs, perm_ref[...], (((1,), (0,)), ((), ())),
        preferred_element_type=jnp.float32,
    )
    tt = tt_ref[0, 0, :].astype(jnp.float32)
    t0 = type_ref[0, :]
    dt = type_ref[1, :] - t0
    x = x + pos_ref[...] + t0[None, :] + tt[:, None] * dt[None, :]
    u = jnp.mean(x, axis=-1, keepdims=True)
    xc = x - u
    v = jnp.mean(xc * xc, axis=-1, keepdims=True)
    y = xc * lax.rsqrt(v + _EPS)
    o_ref[0] = y * lnw_ref[...][None, :] + lnb_ref[...][None, :]


def _ln_body_first(w_ref, perm_ref, tt_ref, pos_ref, type_ref, lnw_ref,
                   lnb_ref, o_ref):
    _ln_body(None, w_ref, perm_ref, tt_ref, pos_ref, type_ref, lnw_ref,
             lnb_ref, o_ref)


def _tc_layernorm_piece(carry, bt, w_rows, perm, tt3, pos_table, type_table,
                        ln_w, ln_b, b0):
    bp, s, h = w_rows.shape
    specs = [
        pl.BlockSpec((1, s, h), lambda i: (i, 0, 0)),
        pl.BlockSpec((h, h), lambda i: (0, 0)),
        pl.BlockSpec((1, 1, s), lambda i, b0=b0: (b0 + i, 0, 0)),
        pl.BlockSpec((s, h), lambda i: (0, 0)),
        pl.BlockSpec((2, h), lambda i: (0, 0)),
        pl.BlockSpec((h,), lambda i: (0,)),
        pl.BlockSpec((h,), lambda i: (0,)),
    ]
    common = dict(
        grid=(bp,),
        out_specs=pl.BlockSpec((1, s, h), lambda i, b0=b0: (b0 + i, 0, 0)),
        out_shape=jax.ShapeDtypeStruct((bt, s, h), jnp.float32),
    )
    if carry is None:
        return pl.pallas_call(_ln_body_first, in_specs=specs, **common)(
            w_rows, perm, tt3, pos_table, type_table, ln_w, ln_b)
    return pl.pallas_call(
        _ln_body,
        in_specs=[pl.BlockSpec(memory_space=pl.ANY)] + specs,
        input_output_aliases={0: 0},
        **common,
    )(carry, w_rows, perm, tt3, pos_table, type_table, ln_w, ln_b)


_PIECES = 4


def kernel(input_ids, token_type_ids, word_table, pos_table, type_table,
           ln_weight, ln_bias):
    b, s = input_ids.shape
    h = word_table.shape[1]
    ids_flat = input_ids.reshape(-1).astype(jnp.int32)
    tt3 = token_type_ids.reshape(b, 1, s).astype(jnp.int32)
    table_u32 = lax.bitcast_convert_type(word_table, jnp.uint32)

    s_idx = jnp.arange(h)
    logical = 32 * (s_idx // 32) + (s_idx // 2) % 16 + 16 * (s_idx % 2)
    perm = (logical[:, None] == jnp.arange(h)[None, :]).astype(jnp.bfloat16)

    bp = b // _PIECES
    npiece = bp * s
    rows = [
        _sc_gather_bf16(table_u32, ids_flat, chunk=64, piece0=p * npiece,
                        npiece=npiece)
        for p in range(_PIECES)
    ]
    out = None
    for p in range(_PIECES):
        w_bf = lax.bitcast_convert_type(rows[p], jnp.bfloat16)
        out = _tc_layernorm_piece(
            out, b, w_bf.reshape(bp, s, h), perm, tt3,
            pos_table, type_table, ln_weight, ln_bias, p * bp)
    return out


# speedup vs baseline: 1.7790x; 1.7790x over previous
import functools

import jax
import jax.numpy as jnp
from jax import lax
from jax.experimental import pallas as pl
from jax.experimental.pallas import tpu as pltpu
from jax.experimental.pallas import tpu_sc as plsc

_EPS = 1e-12


_NUM_CORES = 2
_NUM_SUBCORES = 16
_NUM_WORKERS = _NUM_CORES * _NUM_SUBCORES
_CHUNK = 64
_HALF = _CHUNK // 2


def _sc_gather_bf16(table_u32, ids_flat, piece0, npiece):
    h = table_u32.shape[1]
    tok_per_w = npiece // _NUM_WORKERS
    n_chunks = tok_per_w // _CHUNK
    mesh = plsc.VectorSubcoreMesh(core_axis_name="c", subcore_axis_name="s")

    @functools.partial(
        pl.kernel,
        mesh=mesh,
        out_type=jax.ShapeDtypeStruct((npiece // 2, h), jnp.uint32),
        scratch_types=[
            pltpu.VMEM((tok_per_w,), jnp.int32),
            pltpu.VMEM((_CHUNK, h), jnp.uint32),
            pltpu.VMEM((_CHUNK, h), jnp.uint32),
            pltpu.SemaphoreType.DMA,
            pltpu.SemaphoreType.DMA,
        ],
    )
    def gather_kernel(table_hbm, idx_hbm, out_hbm, idx_v, buf0, buf1, sem0,
                      sem1):
        wid = lax.axis_index("s") * _NUM_CORES + lax.axis_index("c")
        base = wid * tok_per_w
        pltpu.sync_copy(idx_hbm.at[pl.ds(piece0 + base, tok_per_w)], idx_v)

        bufs = (buf0, buf1)
        sems = (sem0, sem1)

        pltpu.async_copy(table_hbm.at[idx_v.at[pl.ds(0, _CHUNK)]], buf0, sem0)

        def body(i, _):
            for p in range(2):
                nxt = i + 1

                @pl.when(jnp.logical_and(nxt % 2 == p, nxt < n_chunks))
                def _():
                    pltpu.async_copy(
                        table_hbm.at[idx_v.at[pl.ds(nxt * _CHUNK, _CHUNK)]],
                        bufs[p],
                        sems[p],
                    )

            for p in range(2):

                @pl.when(i % 2 == p)
                def _():
                    pltpu.make_async_copy(
                        table_hbm.at[idx_v.at[pl.ds(i * _CHUNK, _CHUNK)]],
                        bufs[p],
                        sems[p],
                    ).wait()

                    def row_body(r, _):
                        lo_row = bufs[p].at[r]
                        hi_row = bufs[p].at[r + _HALF]
                        for c in range(0, h, 16):
                            a = lo_row[pl.ds(c, 16)]
                            b = hi_row[pl.ds(c, 16)]
                            lo = (a + 0x8000) >> 16
                            hi = (b + 0x8000) & jnp.uint32(0xFFFF0000)
                            lo_row[pl.ds(c, 16)] = lo | hi
                        return 0

                    lax.fori_loop(0, _HALF, row_body, 0, unroll=2)
                    off = pl.multiple_of((base + i * _CHUNK) // 2, _HALF)
                    pltpu.sync_copy(
                        bufs[p].at[pl.ds(0, _HALF)],
                        out_hbm.at[pl.ds(off, _HALF)],
                    )

            return 0

        lax.fori_loop(0, n_chunks, body, 0)

    return gather_kernel(table_u32, ids_flat)


def _ln_one(x, pos, tt, type_ref, lnw, lnb):
    t0 = type_ref[0, :]
    dt = type_ref[1, :] - t0
    x = x + pos + t0[None, :] + tt[:, None] * dt[None, :]
    u = jnp.mean(x, axis=-1, keepdims=True)
    xc = x - u
    v = jnp.mean(xc * xc, axis=-1, keepdims=True)
    y = xc * lax.rsqrt(v + _EPS)
    return y * lnw[None, :] + lnb[None, :]


def _ln_body(carry_ref, w_ref, tte_ref, tto_ref, pose_ref, poso_ref,
             type_ref, lnw_ref, lnb_ref, o_ref):
    del carry_ref
    w = w_ref[0]
    xe = lax.bitcast_convert_type(w << 16, jnp.float32)
    xo = lax.bitcast_convert_type(w & jnp.uint32(0xFFFF0000), jnp.float32)
    lnw = lnw_ref[...]
    lnb = lnb_ref[...]
    tte = tte_ref[0, 0, :].astype(jnp.float32)
    tto = tto_ref[0, 0, :].astype(jnp.float32)
    ye = _ln_one(xe, pose_ref[...], tte, type_ref, lnw, lnb)
    yo = _ln_one(xo, poso_ref[...], tto, type_ref, lnw, lnb)
    nq = ye.shape[0] // 32
    for q in range(nq):
        o_ref[0, pl.ds(64 * q, 32)] = ye[32 * q:32 * q + 32]
        o_ref[0, pl.ds(64 * q + 32, 32)] = yo[32 * q:32 * q + 32]


def _ln_body_first(w_ref, tte_ref, tto_ref, pose_ref, poso_ref, type_ref,
                   lnw_ref, lnb_ref, o_ref):
    _ln_body(None, w_ref, tte_ref, tto_ref, pose_ref, poso_ref, type_ref,
             lnw_ref, lnb_ref, o_ref)


def _tc_layernorm_piece(carry, bt, w32, tte, tto, pose, poso, type_table,
                        ln_w, ln_b, b0):
    bp, s2, h = w32.shape
    s = 2 * s2
    specs = [
        pl.BlockSpec((1, s2, h), lambda i: (i, 0, 0)),
        pl.BlockSpec((1, 1, s2), lambda i, b0=b0: (b0 + i, 0, 0)),
        pl.BlockSpec((1, 1, s2), lambda i, b0=b0: (b0 + i, 0, 0)),
        pl.BlockSpec((s2, h), lambda i: (0, 0)),
        pl.BlockSpec((s2, h), lambda i: (0, 0)),
        pl.BlockSpec((2, h), lambda i: (0, 0)),
        pl.BlockSpec((h,), lambda i: (0,)),
        pl.BlockSpec((h,), lambda i: (0,)),
    ]
    common = dict(
        grid=(bp,),
        out_specs=pl.BlockSpec((1, s, h), lambda i, b0=b0: (b0 + i, 0, 0)),
        out_shape=jax.ShapeDtypeStruct((bt, s, h), jnp.float32),
    )
    if carry is None:
        return pl.pallas_call(_ln_body_first, in_specs=specs, **common)(
            w32, tte, tto, pose, poso, type_table, ln_w, ln_b)
    return pl.pallas_call(
        _ln_body,
        in_specs=[pl.BlockSpec(memory_space=pl.ANY)] + specs,
        input_output_aliases={0: 0},
        **common,
    )(carry, w32, tte, tto, pose, poso, type_table, ln_w, ln_b)


_PIECES = 4


def kernel(input_ids, token_type_ids, word_table, pos_table, type_table,
           ln_weight, ln_bias):
    b, s = input_ids.shape
    h = word_table.shape[1]
    ids_flat = input_ids.reshape(-1).astype(jnp.int32)
    table_u32 = lax.bitcast_convert_type(word_table, jnp.uint32)

    nq = s // 64
    pos4 = pos_table.reshape(nq, 2, 32, h)
    pose = pos4[:, 0].reshape(s // 2, h)
    poso = pos4[:, 1].reshape(s // 2, h)
    tt4 = token_type_ids.astype(jnp.int32).reshape(b, nq, 2, 32)
    tte = tt4[:, :, 0].reshape(b, 1, s // 2)
    tto = tt4[:, :, 1].reshape(b, 1, s // 2)

    bp = b // _PIECES
    npiece = bp * s
    rows = [
        _sc_gather_bf16(table_u32, ids_flat, piece0=p * npiece,
                        npiece=npiece)
        for p in range(_PIECES)
    ]
    out = None
    for p in range(_PIECES):
        out = _tc_layernorm_piece(
            out, b, rows[p].reshape(bp, s // 2, h), tte, tto, pose, poso,
            type_table, ln_weight, ln_bias, p * bp)
    return out


# speedup vs baseline: 3.5496x; 1.9953x over previous
import functools

import jax
import jax.numpy as jnp
from jax import lax
from jax.experimental import pallas as pl
from jax.experimental.pallas import tpu as pltpu
from jax.experimental.pallas import tpu_sc as plsc

_EPS = 1e-12


_NUM_CORES = 2
_NUM_SUBCORES = 16
_NUM_WORKERS = _NUM_CORES * _NUM_SUBCORES


def _sc_gather(table, ids_flat, chunk, piece0, npiece):
    h = table.shape[1]
    tok_per_w = npiece // _NUM_WORKERS
    n_chunks = tok_per_w // chunk
    mesh = plsc.VectorSubcoreMesh(core_axis_name="c", subcore_axis_name="s")

    @functools.partial(
        pl.kernel,
        mesh=mesh,
        out_type=jax.ShapeDtypeStruct((npiece, h), jnp.float32),
        scratch_types=[
            pltpu.VMEM((tok_per_w,), jnp.int32),
            pltpu.VMEM((chunk, h), jnp.float32),
            pltpu.VMEM((chunk, h), jnp.float32),
            pltpu.SemaphoreType.DMA,
            pltpu.SemaphoreType.DMA,
        ],
    )
    def gather_kernel(table_hbm, idx_hbm, out_hbm, idx_v, buf0, buf1, sem0, sem1):
        wid = lax.axis_index("s") * _NUM_CORES + lax.axis_index("c")
        base = wid * tok_per_w
        pltpu.sync_copy(idx_hbm.at[pl.ds(piece0 + base, tok_per_w)], idx_v)

        bufs = (buf0, buf1)
        sems = (sem0, sem1)

        pltpu.async_copy(table_hbm.at[idx_v.at[pl.ds(0, chunk)]], buf0, sem0)

        def body(i, _):
            for p in range(2):
                nxt = i + 1

                @pl.when(jnp.logical_and(nxt % 2 == p, nxt < n_chunks))
                def _():
                    pltpu.async_copy(
                        table_hbm.at[idx_v.at[pl.ds(nxt * chunk, chunk)]],
                        bufs[p],
                        sems[p],
                    )

            for p in range(2):

                @pl.when(i % 2 == p)
                def _():
                    pltpu.make_async_copy(
                        table_hbm.at[idx_v.at[pl.ds(i * chunk, chunk)]],
                        bufs[p],
                        sems[p],
                    ).wait()
                    pltpu.sync_copy(
                        bufs[p], out_hbm.at[pl.ds(base + i * chunk, chunk)]
                    )

            return 0

        lax.fori_loop(0, n_chunks, body, 0)

    return gather_kernel(table, ids_flat)


def _ln_body(carry_ref, w_ref, tt_ref, pos_ref, type_ref, lnw_ref, lnb_ref,
             o_ref):
    del carry_ref
    x = w_ref[0]
    tt = tt_ref[0, 0, :].astype(jnp.float32)
    t0 = type_ref[0, :]
    dt = type_ref[1, :] - t0
    x = x + pos_ref[...] + t0[None, :] + tt[:, None] * dt[None, :]
    u = jnp.mean(x, axis=-1, keepdims=True)
    xc = x - u
    v = jnp.mean(xc * xc, axis=-1, keepdims=True)
    y = xc * lax.rsqrt(v + _EPS)
    o_ref[0] = y * lnw_ref[...][None, :] + lnb_ref[...][None, :]


def _ln_body_first(w_ref, tt_ref, pos_ref, type_ref, lnw_ref, lnb_ref, o_ref):
    _ln_body(None, w_ref, tt_ref, pos_ref, type_ref, lnw_ref, lnb_ref, o_ref)


def _tc_layernorm_piece(carry, bt, w_rows, tt3, pos_table, type_table, ln_w,
                        ln_b, b0):
    bp, s, h = w_rows.shape
    specs = [
        pl.BlockSpec((1, s, h), lambda i: (i, 0, 0)),
        pl.BlockSpec((1, 1, s), lambda i, b0=b0: (b0 + i, 0, 0)),
        pl.BlockSpec((s, h), lambda i: (0, 0)),
        pl.BlockSpec((2, h), lambda i: (0, 0)),
        pl.BlockSpec((h,), lambda i: (0,)),
        pl.BlockSpec((h,), lambda i: (0,)),
    ]
    common = dict(
        grid=(bp,),
        out_specs=pl.BlockSpec((1, s, h), lambda i, b0=b0: (b0 + i, 0, 0)),
        out_shape=jax.ShapeDtypeStruct((bt, s, h), jnp.float32),
    )
    if carry is None:
        return pl.pallas_call(_ln_body_first, in_specs=specs, **common)(
            w_rows, tt3, pos_table, type_table, ln_w, ln_b)
    return pl.pallas_call(
        _ln_body,
        in_specs=[pl.BlockSpec(memory_space=pl.ANY)] + specs,
        input_output_aliases={0: 0},
        **common,
    )(carry, w_rows, tt3, pos_table, type_table, ln_w, ln_b)


_PIECE_SIZES = (4, 8, 8, 8, 4)


def kernel(input_ids, token_type_ids, word_table, pos_table, type_table,
           ln_weight, ln_bias):
    b, s = input_ids.shape
    h = word_table.shape[1]
    ids_flat = input_ids.reshape(-1).astype(jnp.int32)
    tt3 = token_type_ids.reshape(b, 1, s).astype(jnp.int32)

    offs = [sum(_PIECE_SIZES[:p]) for p in range(len(_PIECE_SIZES))]
    rows = [
        _sc_gather(word_table, ids_flat, chunk=64, piece0=b0 * s,
                   npiece=bp * s)
        for b0, bp in zip(offs, _PIECE_SIZES)
    ]
    out = None
    for r, b0, bp in zip(rows, offs, _PIECE_SIZES):
        out = _tc_layernorm_piece(
            out, b, r.reshape(bp, s, h), tt3,
            pos_table, type_table, ln_weight, ln_bias, b0)
    return out


# speedup vs baseline: 3.6406x; 1.0256x over previous
import functools

import jax
import jax.numpy as jnp
from jax import lax
from jax.experimental import pallas as pl
from jax.experimental.pallas import tpu as pltpu
from jax.experimental.pallas import tpu_sc as plsc

_EPS = 1e-12


_NUM_CORES = 2
_NUM_SUBCORES = 16
_NUM_WORKERS = _NUM_CORES * _NUM_SUBCORES


def _sc_gather(table, ids_flat, chunk, piece0, npiece):
    h = table.shape[1]
    tok_per_w = npiece // _NUM_WORKERS
    n_chunks = tok_per_w // chunk
    mesh = plsc.VectorSubcoreMesh(core_axis_name="c", subcore_axis_name="s")

    @functools.partial(
        pl.kernel,
        mesh=mesh,
        out_type=jax.ShapeDtypeStruct((npiece, h), jnp.float32),
        scratch_types=[
            pltpu.VMEM((tok_per_w,), jnp.int32),
            pltpu.VMEM((chunk, h), jnp.float32),
            pltpu.VMEM((chunk, h), jnp.float32),
            pltpu.SemaphoreType.DMA,
            pltpu.SemaphoreType.DMA,
        ],
    )
    def gather_kernel(table_hbm, idx_hbm, out_hbm, idx_v, buf0, buf1, sem0, sem1):
        wid = lax.axis_index("s") * _NUM_CORES + lax.axis_index("c")
        base = wid * tok_per_w
        pltpu.sync_copy(idx_hbm.at[pl.ds(piece0 + base, tok_per_w)], idx_v)

        bufs = (buf0, buf1)
        sems = (sem0, sem1)

        pltpu.async_copy(table_hbm.at[idx_v.at[pl.ds(0, chunk)]], buf0, sem0)

        def body(i, _):
            for p in range(2):
                nxt = i + 1

                @pl.when(jnp.logical_and(nxt % 2 == p, nxt < n_chunks))
                def _():
                    pltpu.async_copy(
                        table_hbm.at[idx_v.at[pl.ds(nxt * chunk, chunk)]],
                        bufs[p],
                        sems[p],
                    )

            for p in range(2):

                @pl.when(i % 2 == p)
                def _():
                    pltpu.make_async_copy(
                        table_hbm.at[idx_v.at[pl.ds(i * chunk, chunk)]],
                        bufs[p],
                        sems[p],
                    ).wait()
                    pltpu.sync_copy(
                        bufs[p], out_hbm.at[pl.ds(base + i * chunk, chunk)]
                    )

            return 0

        lax.fori_loop(0, n_chunks, body, 0)

    return gather_kernel(table, ids_flat)


def _ln_body(carry_ref, w_ref, tt_ref, pos_ref, type_ref, lnw_ref, lnb_ref,
             o_ref):
    del carry_ref
    x = w_ref[0]
    tt = tt_ref[0, 0, :].astype(jnp.float32)
    t0 = type_ref[0, :]
    dt = type_ref[1, :] - t0
    x = x + pos_ref[...] + t0[None, :] + tt[:, None] * dt[None, :]
    u = jnp.mean(x, axis=-1, keepdims=True)
    xc = x - u
    v = jnp.mean(xc * xc, axis=-1, keepdims=True)
    y = xc * lax.rsqrt(v + _EPS)
    o_ref[0] = y * lnw_ref[...][None, :] + lnb_ref[...][None, :]


def _ln_body_first(w_ref, tt_ref, pos_ref, type_ref, lnw_ref, lnb_ref, o_ref):
    _ln_body(None, w_ref, tt_ref, pos_ref, type_ref, lnw_ref, lnb_ref, o_ref)


def _tc_layernorm_piece(carry, bt, w_rows, tt3, pos_table, type_table, ln_w,
                        ln_b, b0):
    bp, s, h = w_rows.shape
    specs = [
        pl.BlockSpec((1, s, h), lambda i: (i, 0, 0)),
        pl.BlockSpec((1, 1, s), lambda i, b0=b0: (b0 + i, 0, 0)),
        pl.BlockSpec((s, h), lambda i: (0, 0)),
        pl.BlockSpec((2, h), lambda i: (0, 0)),
        pl.BlockSpec((h,), lambda i: (0,)),
        pl.BlockSpec((h,), lambda i: (0,)),
    ]
    common = dict(
        grid=(bp,),
        out_specs=pl.BlockSpec((1, s, h), lambda i, b0=b0: (b0 + i, 0, 0)),
        out_shape=jax.ShapeDtypeStruct((bt, s, h), jnp.float32),
    )
    if carry is None:
        return pl.pallas_call(_ln_body_first, in_specs=specs, **common)(
            w_rows, tt3, pos_table, type_table, ln_w, ln_b)
    return pl.pallas_call(
        _ln_body,
        in_specs=[pl.BlockSpec(memory_space=pl.ANY)] + specs,
        input_output_aliases={0: 0},
        **common,
    )(carry, w_rows, tt3, pos_table, type_table, ln_w, ln_b)


_PIECE_SIZES = (8, 12, 12)


def kernel(input_ids, token_type_ids, word_table, pos_table, type_table,
           ln_weight, ln_bias):
    b, s = input_ids.shape
    h = word_table.shape[1]
    ids_flat = input_ids.reshape(-1).astype(jnp.int32)
    tt3 = token_type_ids.reshape(b, 1, s).astype(jnp.int32)

    offs = [sum(_PIECE_SIZES[:p]) for p in range(len(_PIECE_SIZES))]
    rows = [
        _sc_gather(word_table, ids_flat, chunk=64, piece0=b0 * s,
                   npiece=bp * s)
        for b0, bp in zip(offs, _PIECE_SIZES)
    ]
    out = None
    for r, b0, bp in zip(rows, offs, _PIECE_SIZES):
        out = _tc_layernorm_piece(
            out, b, r.reshape(bp, s, h), tt3,
            pos_table, type_table, ln_weight, ln_bias, b0)
    return out
